# baseline (speedup 1.0000x reference)
import jax
import jax.numpy as jnp
from jax.experimental import pallas as pl
from jax.experimental.pallas import tpu as pltpu

_B = 8
_T = 4
_P = 1025
_H = 1280
_PH = _P * _H


def _body(ids_ref, h_ref, t_ref, e_ref, gate_ref, o_ref):
    g = jnp.tanh(gate_ref[0])
    tt = t_ref[...].reshape(_P, _H)
    o_ref[...] = (h_ref[...]
                  + (1.0 - g) * e_ref[...]
                  + g * tt[None, None])


def kernel(hidden_state, aspect_ratio_ids, gate, embedding, tile_embedding_weight):
    ids = aspect_ratio_ids.astype(jnp.int32)
    tv = tile_embedding_weight.reshape(9, 1, _T * _PH)

    grid_spec = pltpu.PrefetchScalarGridSpec(
        num_scalar_prefetch=1,
        grid=(_B, _T),
        in_specs=[
            pl.BlockSpec((1, 1, _P, _H), lambda b, t, ids: (b, t, 0, 0)),
            pl.BlockSpec((1, 1, _PH), lambda b, t, ids: (ids[b], 0, t)),
            pl.BlockSpec((_P, _H), lambda b, t, ids: (0, 0)),
            pl.BlockSpec(memory_space=pltpu.SMEM),
        ],
        out_specs=pl.BlockSpec((1, 1, _P, _H), lambda b, t, ids: (b, t, 0, 0)),
    )
    out = pl.pallas_call(
        _body,
        grid_spec=grid_spec,
        out_shape=jax.ShapeDtypeStruct((_B, _T, _P, _H), jnp.float32),
        compiler_params=pltpu.CompilerParams(
            dimension_semantics=("arbitrary", "arbitrary"),
        ),
    )(ids, hidden_state, tv, embedding, gate)
    return out
